# baseline (speedup 1.0000x reference)
import jax
import jax.numpy as jnp
from jax.experimental import pallas as pl

_SORT8 = [(0, 1), (2, 3), (4, 5), (6, 7),
          (0, 2), (1, 3), (4, 6), (5, 7),
          (1, 2), (5, 6),
          (0, 4), (1, 5), (2, 6), (3, 7),
          (2, 4), (3, 5),
          (1, 2), (3, 4), (5, 6)]

_BITONIC8 = [(0, 4), (1, 5), (2, 6), (3, 7),
             (0, 2), (1, 3), (4, 6), (5, 7),
             (0, 1), (2, 3), (4, 5), (6, 7)]


def _ce(v, pairs):
    for i, j in pairs:
        hi = jnp.maximum(v[i], v[j])
        lo = jnp.minimum(v[i], v[j])
        v[i], v[j] = hi, lo
    return v


def _merge_top8(a, b):
    m = [jnp.maximum(a[k], b[7 - k]) for k in range(8)]
    return _ce(m, _BITONIC8)


def _topk_mask_body(x_ref, o_ref):
    x = x_ref[0]
    g = x.reshape(12, 8, x.shape[1], x.shape[2])
    v = [g[:, k] for k in range(8)]
    v = _ce(v, _SORT8)

    a = [u.reshape(6, 2, u.shape[1], u.shape[2])[:, 0] for u in v]
    b = [u.reshape(6, 2, u.shape[1], u.shape[2])[:, 1] for u in v]
    v = _merge_top8(a, b)
    a = [u.reshape(3, 2, u.shape[1], u.shape[2])[:, 0] for u in v]
    b = [u.reshape(3, 2, u.shape[1], u.shape[2])[:, 1] for u in v]
    v = _merge_top8(a, b)
    a = [u[0:1] for u in v]
    b = [u[1:2] for u in v]
    c = [u[2:3] for u in v]
    ab = _merge_top8(a, b)
    m = [jnp.maximum(ab[k], c[7 - k]) for k in range(8)]
    t = m[0]
    for k in range(1, 8):
        t = jnp.minimum(t, m[k])

    o_ref[0] = (x >= t).astype(jnp.float32)


def kernel(activations):
    B, C, H, W = activations.shape
    out = pl.pallas_call(
        _topk_mask_body,
        grid=(B,),
        in_specs=[pl.BlockSpec((1, C, H, W), lambda i: (i, 0, 0, 0))],
        out_specs=pl.BlockSpec((1, C, H, W), lambda i: (i, 0, 0, 0)),
        out_shape=jax.ShapeDtypeStruct((B, C, H, W), jnp.float32),
    )(activations)
    return out


# speedup vs baseline: 1.7758x; 1.7758x over previous
import jax
import jax.numpy as jnp
from jax.experimental import pallas as pl
from jax.experimental.pallas import tpu as pltpu
from jax.experimental.pallas import tpu_sc as plsc

_SORT8 = [(0, 1), (2, 3), (4, 5), (6, 7),
          (0, 2), (1, 3), (4, 6), (5, 7),
          (1, 2), (5, 6),
          (0, 4), (1, 5), (2, 6), (3, 7),
          (2, 4), (3, 5),
          (1, 2), (3, 4), (5, 6)]

_BITONIC8 = [(0, 4), (1, 5), (2, 6), (3, 7),
             (0, 2), (1, 3), (4, 6), (5, 7),
             (0, 1), (2, 3), (4, 5), (6, 7)]

_WBLK = 64


def _merge_top8(a, b, resort=True):
    m = [jnp.maximum(a[k], b[7 - k]) for k in range(8)]
    if resort:
        for i, j in _BITONIC8:
            hi = jnp.maximum(m[i], m[j])
            lo = jnp.minimum(m[i], m[j])
            m[i], m[j] = hi, lo
    return m


def _sc_block_body(x_vmem, o_vmem):
    n_c = x_vmem.shape[0]
    for col in range(_WBLK // 16):
        s = pl.ds(col * 16, 16)
        rows = [x_vmem[r, s] for r in range(n_c)]
        lists = []
        for g in range(12):
            v = [rows[g * 8 + k] for k in range(8)]
            for i, j in _SORT8:
                hi = jnp.maximum(v[i], v[j])
                lo = jnp.minimum(v[i], v[j])
                v[i], v[j] = hi, lo
            lists.append(v)
        while len(lists) > 2:
            nxt = [_merge_top8(lists[2 * i], lists[2 * i + 1])
                   for i in range(len(lists) // 2)]
            if len(lists) % 2:
                nxt.append(lists[-1])
            lists = nxt
        m = _merge_top8(lists[0], lists[1], resort=False)
        t = m[0]
        for k in range(1, 8):
            t = jnp.minimum(t, m[k])
        one = jnp.full((16,), 1.0, dtype=jnp.float32)
        zero = jnp.zeros((16,), dtype=jnp.float32)
        for r in range(n_c):
            o_vmem[r, s] = jnp.where(rows[r] >= t, one, zero)


def kernel(activations):
    B, C, H, W = activations.shape
    N = H * W * B
    xt = jnp.transpose(activations, (1, 2, 3, 0)).reshape(C, N)
    mesh = plsc.VectorSubcoreMesh(core_axis_name="c", subcore_axis_name="s")

    @pl.kernel(out_type=jax.ShapeDtypeStruct((C, N), jnp.float32), mesh=mesh)
    def run(x_hbm, o_hbm):
        pltpu.emit_pipeline(
            _sc_block_body,
            grid=(N // _WBLK,),
            in_specs=[pl.BlockSpec((C, _WBLK), lambda i: (0, i))],
            out_specs=[pl.BlockSpec((C, _WBLK), lambda i: (0, i))],
            core_axis_name=("c", "s"),
            dimension_semantics=(pltpu.PARALLEL,),
        )(x_hbm, o_hbm)

    out = run(xt)
    return jnp.transpose(out.reshape(C, H, W, B), (3, 0, 1, 2))


# speedup vs baseline: 13.0594x; 7.3541x over previous
import jax
import jax.numpy as jnp
from jax.experimental import pallas as pl
from jax.experimental.pallas import tpu as pltpu

_SORT8 = [(0, 1), (2, 3), (4, 5), (6, 7),
          (0, 2), (1, 3), (4, 6), (5, 7),
          (1, 2), (5, 6),
          (0, 4), (1, 5), (2, 6), (3, 7),
          (2, 4), (3, 5),
          (1, 2), (3, 4), (5, 6)]

_BITONIC8 = [(0, 4), (1, 5), (2, 6), (3, 7),
             (0, 2), (1, 3), (4, 6), (5, 7),
             (0, 1), (2, 3), (4, 5), (6, 7)]


def _ce(v, pairs):
    for i, j in pairs:
        hi = jnp.maximum(v[i], v[j])
        lo = jnp.minimum(v[i], v[j])
        v[i], v[j] = hi, lo
    return v


def _merge_top8(a, b):
    m = [jnp.maximum(a[k], b[7 - k]) for k in range(8)]
    return _ce(m, _BITONIC8)


def _topk_mask_body(x_ref, o_ref):
    n_c, n_h, n_w, n_b = x_ref.shape
    x = x_ref[...].reshape(n_c, n_h * n_w, n_b)
    g = x.reshape(12, 8, x.shape[1], x.shape[2])
    v = [g[:, k] for k in range(8)]
    v = _ce(v, _SORT8)

    a = [u.reshape(6, 2, u.shape[1], u.shape[2])[:, 0] for u in v]
    b = [u.reshape(6, 2, u.shape[1], u.shape[2])[:, 1] for u in v]
    v = _merge_top8(a, b)
    a = [u.reshape(3, 2, u.shape[1], u.shape[2])[:, 0] for u in v]
    b = [u.reshape(3, 2, u.shape[1], u.shape[2])[:, 1] for u in v]
    v = _merge_top8(a, b)
    a = [u[0:1] for u in v]
    b = [u[1:2] for u in v]
    c = [u[2:3] for u in v]
    ab = _merge_top8(a, b)
    m = [jnp.maximum(ab[k], c[7 - k]) for k in range(8)]
    t = m[0]
    for k in range(1, 8):
        t = jnp.minimum(t, m[k])

    o_ref[...] = (x >= t).astype(jnp.float32).reshape(n_c, n_h, n_w, n_b)


def kernel(activations):
    B, C, H, W = activations.shape
    xt = jnp.transpose(activations, (1, 2, 3, 0))
    HB = 4
    out = pl.pallas_call(
        _topk_mask_body,
        grid=(H // HB,),
        in_specs=[pl.BlockSpec((C, HB, W, B), lambda i: (0, i, 0, 0))],
        out_specs=pl.BlockSpec((C, HB, W, B), lambda i: (0, i, 0, 0)),
        out_shape=jax.ShapeDtypeStruct((C, H, W, B), jnp.float32),
        compiler_params=pltpu.CompilerParams(
            dimension_semantics=("parallel",)),
    )(xt)
    return jnp.transpose(out, (3, 0, 1, 2))
